# Initial kernel scaffold; baseline (speedup 1.0000x reference)
import jax
import jax.numpy as jnp
from jax.experimental import pallas as pl


def kernel(x, edge_index, edge_index_reverse, W1, bc1, W2, bc2, w11, w12, b1, w21, w22, b2):
    raise NotImplementedError("write your pallas kernel here")


# speedup vs baseline: 19.2845x; 19.2845x over previous
import functools

import jax
import jax.numpy as jnp
from jax import lax
from jax.experimental import pallas as pl
from jax.experimental.pallas import tpu as pltpu
from jax.experimental.pallas import tpu_sc as plsc

N = 10000
D = 128
E = 320000

NT = 16
EPT = E // NT
CHUNK = 128
CPT = 160
EPTP = CPT * CHUNK
IG = 8
NG = CPT // IG
NPAD = 10240
RACC = NPAD // NT
BLK = 1024
GRID = NPAD // BLK

_mesh = plsc.VectorSubcoreMesh(core_axis_name="c", subcore_axis_name="s")


@functools.partial(
    pl.kernel,
    out_type=(jax.ShapeDtypeStruct((NPAD,), jnp.float32),
              jax.ShapeDtypeStruct((NPAD,), jnp.float32)),
    mesh=_mesh,
    scratch_types=(
        pltpu.VMEM((IG, CHUNK), jnp.int32),
        pltpu.VMEM((CHUNK,), jnp.float32),
        pltpu.VMEM_SHARED((NPAD,), jnp.float32),
    ),
)
def _deg_kernel(dstf, dstr, outf, outr, dst_v, ones_v, acc):
    cid = lax.axis_index("c")
    sid = lax.axis_index("s")

    def run(dst3, out):
        for k in range(CHUNK // 16):
            ones_v[pl.ds(k * 16, 16)] = jnp.zeros((16,), jnp.float32)
        base = sid * RACC
        for c in range(RACC // CHUNK):
            pltpu.sync_copy(ones_v, acc.at[pl.ds(base + c * CHUNK, CHUNK)])
        for k in range(CHUNK // 16):
            ones_v[pl.ds(k * 16, 16)] = jnp.ones((16,), jnp.float32)
        plsc.subcore_barrier()

        def group(g, carry):
            pltpu.sync_copy(dst3.at[sid, pl.ds(g * IG, IG)], dst_v)

            def body(j, c2):
                pltpu.sync_copy(ones_v, acc.at[dst_v.at[j]], add=True)
                return c2

            return lax.fori_loop(0, IG, body, carry)

        lax.fori_loop(0, NG, group, 0)
        plsc.subcore_barrier()
        pltpu.sync_copy(acc.at[pl.ds(base, RACC)], out.at[pl.ds(base, RACC)])

    @pl.when(cid == 0)
    def _():
        run(dstf, outf)

    @pl.when(cid == 1)
    def _():
        run(dstr, outr)


@functools.partial(
    pl.kernel,
    out_type=(jax.ShapeDtypeStruct((NPAD, D), jnp.float32),
              jax.ShapeDtypeStruct((NPAD, D), jnp.float32)),
    mesh=_mesh,
    scratch_types=(
        pltpu.VMEM((IG, CHUNK), jnp.int32),
        pltpu.VMEM((IG, CHUNK), jnp.int32),
        pltpu.VMEM((CHUNK, D), jnp.float32),
        pltpu.VMEM_SHARED((NPAD, D), jnp.float32),
        pltpu.SemaphoreType.DMA,
    ),
)
def _agg_kernel(hsf, hsr, srcf, dstf, srcr, dstr, outf, outr,
                src_v, dst_v, rows_v, acc, sem):
    cid = lax.axis_index("c")
    sid = lax.axis_index("s")

    def run(hs, src3, dst3, out):
        def zrow(r, carry):
            for k in range(D // 16):
                rows_v[r, pl.ds(k * 16, 16)] = jnp.zeros((16,), jnp.float32)
            return carry

        lax.fori_loop(0, CHUNK, zrow, 0)
        base = sid * RACC
        for c in range(RACC // CHUNK):
            pltpu.sync_copy(rows_v, acc.at[pl.ds(base + c * CHUNK, CHUNK)])
        plsc.subcore_barrier()

        def group(g, carry):
            pltpu.sync_copy(src3.at[sid, pl.ds(g * IG, IG)], src_v)
            pltpu.sync_copy(dst3.at[sid, pl.ds(g * IG, IG)], dst_v)

            def body(j, c2):
                pltpu.async_copy(hs.at[src_v.at[j]], rows_v, sem).wait()
                pltpu.sync_copy(rows_v, acc.at[dst_v.at[j]], add=True)
                return c2

            return lax.fori_loop(0, IG, body, carry)

        lax.fori_loop(0, NG, group, 0)
        plsc.subcore_barrier()
        pltpu.sync_copy(acc.at[pl.ds(base, RACC)], out.at[pl.ds(base, RACC)])

    @pl.when(cid == 0)
    def _():
        run(hsf, srcf, dstf, outf)

    @pl.when(cid == 1)
    def _():
        run(hsr, srcr, dstr, outr)


def _tc_pre_body(x_ref, w1_ref, degf_ref, degr_ref,
                 hsf_ref, hsr_ref, dinvf_ref, dinvr_ref):
    h1 = jnp.dot(x_ref[...], w1_ref[...], preferred_element_type=jnp.float32)
    dinvf = lax.rsqrt(degf_ref[...] + 1.0)
    dinvr = lax.rsqrt(degr_ref[...] + 1.0)
    hsf_ref[...] = h1 * dinvf
    hsr_ref[...] = h1 * dinvr
    dinvf_ref[...] = dinvf
    dinvr_ref[...] = dinvr


def _tc_mid_body(aggf_ref, aggr_ref, hsf_ref, hsr_ref, dinvf_ref, dinvr_ref,
                 bc1_ref, w11t_ref, w12t_ref, b1_ref, w2_ref,
                 hs2f_ref, hs2r_ref):
    dinvf = dinvf_ref[...]
    dinvr = dinvr_ref[...]
    c11 = jax.nn.relu(dinvf * (aggf_ref[...] + hsf_ref[...]) + bc1_ref[...])
    c12 = jax.nn.relu(dinvr * (aggr_ref[...] + hsr_ref[...]) + bc1_ref[...])
    g = jax.nn.sigmoid(
        jnp.dot(c11, w11t_ref[...], preferred_element_type=jnp.float32)
        + jnp.dot(c12, w12t_ref[...], preferred_element_type=jnp.float32)
        + b1_ref[...])
    h = g * c11 + (1.0 - g) * c12
    h2 = jnp.dot(h, w2_ref[...], preferred_element_type=jnp.float32)
    hs2f_ref[...] = h2 * dinvf
    hs2r_ref[...] = h2 * dinvr


def _tc_post_body(aggf_ref, aggr_ref, hsf_ref, hsr_ref, dinvf_ref, dinvr_ref,
                  bc2_ref, w21t_ref, w22t_ref, b2_ref, out_ref):
    c21 = jax.nn.relu(dinvf_ref[...] * (aggf_ref[...] + hsf_ref[...]) + bc2_ref[...])
    c22 = jax.nn.relu(dinvr_ref[...] * (aggr_ref[...] + hsr_ref[...]) + bc2_ref[...])
    g2 = jax.nn.sigmoid(
        jnp.dot(c21, w21t_ref[...], preferred_element_type=jnp.float32)
        + jnp.dot(c22, w22t_ref[...], preferred_element_type=jnp.float32)
        + b2_ref[...])
    out_ref[...] = g2 * c21 + (1.0 - g2) * c22


_row_spec = pl.BlockSpec((BLK, D), lambda b: (b, 0))
_col_spec = pl.BlockSpec((BLK, 1), lambda b: (b, 0))
_w_spec = pl.BlockSpec((D, D), lambda b: (0, 0))
_b_spec = pl.BlockSpec((1, D), lambda b: (0, 0))

_tc_pre = pl.pallas_call(
    _tc_pre_body,
    grid=(GRID,),
    in_specs=[_row_spec, _w_spec, _col_spec, _col_spec],
    out_specs=[_row_spec, _row_spec, _col_spec, _col_spec],
    out_shape=[jax.ShapeDtypeStruct((NPAD, D), jnp.float32),
               jax.ShapeDtypeStruct((NPAD, D), jnp.float32),
               jax.ShapeDtypeStruct((NPAD, 1), jnp.float32),
               jax.ShapeDtypeStruct((NPAD, 1), jnp.float32)],
)

_tc_mid = pl.pallas_call(
    _tc_mid_body,
    grid=(GRID,),
    in_specs=[_row_spec, _row_spec, _row_spec, _row_spec, _col_spec, _col_spec,
              _b_spec, _w_spec, _w_spec, _b_spec, _w_spec],
    out_specs=[_row_spec, _row_spec],
    out_shape=[jax.ShapeDtypeStruct((NPAD, D), jnp.float32),
               jax.ShapeDtypeStruct((NPAD, D), jnp.float32)],
)

_tc_post = pl.pallas_call(
    _tc_post_body,
    grid=(GRID,),
    in_specs=[_row_spec, _row_spec, _row_spec, _row_spec, _col_spec, _col_spec,
              _b_spec, _w_spec, _w_spec, _b_spec],
    out_specs=_row_spec,
    out_shape=jax.ShapeDtypeStruct((NPAD, D), jnp.float32),
)


def _pad_edges(edge_index):
    npad_e = EPTP - EPT
    src = edge_index[0].reshape(NT, EPT)
    dst = edge_index[1].reshape(NT, EPT)
    fill = jnp.arange(NT * npad_e, dtype=jnp.int32).reshape(NT, npad_e)
    src_fill = (fill * 97) % N
    dst_fill = N + (fill % (NPAD - N))
    src3 = jnp.concatenate([src, src_fill], axis=1).reshape(NT, CPT, CHUNK)
    dst3 = jnp.concatenate([dst, dst_fill], axis=1).reshape(NT, CPT, CHUNK)
    return src3, dst3


@jax.jit
def kernel(x, edge_index, edge_index_reverse, W1, bc1, W2, bc2,
           w11, w12, b1, w21, w22, b2):
    srcf3, dstf3 = _pad_edges(edge_index)
    srcr3, dstr3 = _pad_edges(edge_index_reverse)

    x_pad = jnp.zeros((NPAD, D), jnp.float32).at[:N].set(x)

    degf, degr = _deg_kernel(dstf3, dstr3)
    degf = degf.reshape(NPAD, 1)
    degr = degr.reshape(NPAD, 1)

    hs1f, hs1r, dinvf, dinvr = _tc_pre(x_pad, W1, degf, degr)
    agg1f, agg1r = _agg_kernel(hs1f, hs1r, srcf3, dstf3, srcr3, dstr3)
    hs2f, hs2r = _tc_mid(agg1f, agg1r, hs1f, hs1r, dinvf, dinvr,
                         bc1.reshape(1, D), w11.T, w12.T, b1.reshape(1, D), W2)
    agg2f, agg2r = _agg_kernel(hs2f, hs2r, srcf3, dstf3, srcr3, dstr3)
    out = _tc_post(agg2f, agg2r, hs2f, hs2r, dinvf, dinvr,
                   bc2.reshape(1, D), w21.T, w22.T, b2.reshape(1, D))
    return out[:N]


# speedup vs baseline: 23.9706x; 1.2430x over previous
import functools

import jax
import jax.numpy as jnp
from jax import lax
from jax.experimental import pallas as pl
from jax.experimental.pallas import tpu as pltpu
from jax.experimental.pallas import tpu_sc as plsc

N = 10000
D = 128
E = 320000

NT = 16
EPT = E // NT
CHUNK = 128
CPT = 160
EPTP = CPT * CHUNK
IG = 8
NG = CPT // IG
NPAD = 10240
RACC = NPAD // NT
BLK = 1024
GRID = NPAD // BLK

_mesh = plsc.VectorSubcoreMesh(core_axis_name="c", subcore_axis_name="s")


@functools.partial(
    pl.kernel,
    out_type=(jax.ShapeDtypeStruct((NPAD,), jnp.float32),
              jax.ShapeDtypeStruct((NPAD,), jnp.float32)),
    mesh=_mesh,
    scratch_types=(
        pltpu.VMEM((IG, CHUNK), jnp.int32),
        pltpu.VMEM((CHUNK,), jnp.float32),
        pltpu.VMEM_SHARED((NPAD,), jnp.float32),
    ),
)
def _deg_kernel(dstf, dstr, outf, outr, dst_v, ones_v, acc):
    cid = lax.axis_index("c")
    sid = lax.axis_index("s")

    def run(dst3, out):
        for k in range(CHUNK // 16):
            ones_v[pl.ds(k * 16, 16)] = jnp.zeros((16,), jnp.float32)
        base = sid * RACC
        for c in range(RACC // CHUNK):
            pltpu.sync_copy(ones_v, acc.at[pl.ds(base + c * CHUNK, CHUNK)])
        for k in range(CHUNK // 16):
            ones_v[pl.ds(k * 16, 16)] = jnp.ones((16,), jnp.float32)
        plsc.subcore_barrier()

        def group(g, carry):
            pltpu.sync_copy(dst3.at[sid, pl.ds(g * IG, IG)], dst_v)

            def body(j, c2):
                pltpu.sync_copy(ones_v, acc.at[dst_v.at[j]], add=True)
                return c2

            return lax.fori_loop(0, IG, body, carry)

        lax.fori_loop(0, NG, group, 0)
        plsc.subcore_barrier()
        pltpu.sync_copy(acc.at[pl.ds(base, RACC)], out.at[pl.ds(base, RACC)])

    @pl.when(cid == 0)
    def _():
        run(dstf, outf)

    @pl.when(cid == 1)
    def _():
        run(dstr, outr)


@functools.partial(
    pl.kernel,
    out_type=(jax.ShapeDtypeStruct((NPAD, D), jnp.float32),
              jax.ShapeDtypeStruct((NPAD, D), jnp.float32)),
    mesh=_mesh,
    scratch_types=(
        pltpu.VMEM((IG, CHUNK), jnp.int32),
        pltpu.VMEM((IG, CHUNK), jnp.int32),
        pltpu.VMEM((CHUNK, D), jnp.float32),
        pltpu.VMEM((CHUNK, D), jnp.float32),
        pltpu.VMEM_SHARED((NPAD, D), jnp.float32),
        pltpu.SemaphoreType.DMA,
        pltpu.SemaphoreType.DMA,
    ),
)
def _agg_kernel(hsf, hsr, srcf, dstf, srcr, dstr, outf, outr,
                src_v, dst_v, rows0, rows1, acc, gsem, ssem):
    cid = lax.axis_index("c")
    sid = lax.axis_index("s")

    def run(hs, src3, dst3, out):
        bufs = (rows0, rows1)
        def zrow(r, carry):
            for k in range(D // 16):
                rows0[r, pl.ds(k * 16, 16)] = jnp.zeros((16,), jnp.float32)
            return carry

        lax.fori_loop(0, CHUNK, zrow, 0)
        base = sid * RACC
        for c in range(RACC // CHUNK):
            pltpu.sync_copy(rows0, acc.at[pl.ds(base + c * CHUNK, CHUNK)])
        plsc.subcore_barrier()

        def group(g, carry):
            pltpu.sync_copy(src3.at[sid, pl.ds(g * IG, IG)], src_v)
            pltpu.sync_copy(dst3.at[sid, pl.ds(g * IG, IG)], dst_v)
            gd = [None] * IG
            sd = [None] * IG
            gd[0] = pltpu.async_copy(hs.at[src_v.at[0]], bufs[0], gsem)
            for j in range(IG):
                b = bufs[j % 2]
                gd[j].wait()
                sd[j] = pltpu.async_copy(b, acc.at[dst_v.at[j]], ssem, add=True)
                if j + 1 < IG:
                    if j >= 1:
                        sd[j - 1].wait()
                    gd[j + 1] = pltpu.async_copy(
                        hs.at[src_v.at[j + 1]], bufs[(j + 1) % 2], gsem)
            sd[IG - 2].wait()
            sd[IG - 1].wait()
            return carry

        lax.fori_loop(0, NG, group, 0)
        plsc.subcore_barrier()
        pltpu.sync_copy(acc.at[pl.ds(base, RACC)], out.at[pl.ds(base, RACC)])

    @pl.when(cid == 0)
    def _():
        run(hsf, srcf, dstf, outf)

    @pl.when(cid == 1)
    def _():
        run(hsr, srcr, dstr, outr)


def _tc_pre_body(x_ref, w1_ref, degf_ref, degr_ref,
                 hsf_ref, hsr_ref, dinvf_ref, dinvr_ref):
    h1 = jnp.dot(x_ref[...], w1_ref[...], preferred_element_type=jnp.float32)
    dinvf = lax.rsqrt(degf_ref[...] + 1.0)
    dinvr = lax.rsqrt(degr_ref[...] + 1.0)
    hsf_ref[...] = h1 * dinvf
    hsr_ref[...] = h1 * dinvr
    dinvf_ref[...] = dinvf
    dinvr_ref[...] = dinvr


def _tc_mid_body(aggf_ref, aggr_ref, hsf_ref, hsr_ref, dinvf_ref, dinvr_ref,
                 bc1_ref, w11t_ref, w12t_ref, b1_ref, w2_ref,
                 hs2f_ref, hs2r_ref):
    dinvf = dinvf_ref[...]
    dinvr = dinvr_ref[...]
    c11 = jax.nn.relu(dinvf * (aggf_ref[...] + hsf_ref[...]) + bc1_ref[...])
    c12 = jax.nn.relu(dinvr * (aggr_ref[...] + hsr_ref[...]) + bc1_ref[...])
    g = jax.nn.sigmoid(
        jnp.dot(c11, w11t_ref[...], preferred_element_type=jnp.float32)
        + jnp.dot(c12, w12t_ref[...], preferred_element_type=jnp.float32)
        + b1_ref[...])
    h = g * c11 + (1.0 - g) * c12
    h2 = jnp.dot(h, w2_ref[...], preferred_element_type=jnp.float32)
    hs2f_ref[...] = h2 * dinvf
    hs2r_ref[...] = h2 * dinvr


def _tc_post_body(aggf_ref, aggr_ref, hsf_ref, hsr_ref, dinvf_ref, dinvr_ref,
                  bc2_ref, w21t_ref, w22t_ref, b2_ref, out_ref):
    c21 = jax.nn.relu(dinvf_ref[...] * (aggf_ref[...] + hsf_ref[...]) + bc2_ref[...])
    c22 = jax.nn.relu(dinvr_ref[...] * (aggr_ref[...] + hsr_ref[...]) + bc2_ref[...])
    g2 = jax.nn.sigmoid(
        jnp.dot(c21, w21t_ref[...], preferred_element_type=jnp.float32)
        + jnp.dot(c22, w22t_ref[...], preferred_element_type=jnp.float32)
        + b2_ref[...])
    out_ref[...] = g2 * c21 + (1.0 - g2) * c22


_row_spec = pl.BlockSpec((BLK, D), lambda b: (b, 0))
_col_spec = pl.BlockSpec((BLK, 1), lambda b: (b, 0))
_w_spec = pl.BlockSpec((D, D), lambda b: (0, 0))
_b_spec = pl.BlockSpec((1, D), lambda b: (0, 0))

_tc_pre = pl.pallas_call(
    _tc_pre_body,
    grid=(GRID,),
    in_specs=[_row_spec, _w_spec, _col_spec, _col_spec],
    out_specs=[_row_spec, _row_spec, _col_spec, _col_spec],
    out_shape=[jax.ShapeDtypeStruct((NPAD, D), jnp.float32),
               jax.ShapeDtypeStruct((NPAD, D), jnp.float32),
               jax.ShapeDtypeStruct((NPAD, 1), jnp.float32),
               jax.ShapeDtypeStruct((NPAD, 1), jnp.float32)],
)

_tc_mid = pl.pallas_call(
    _tc_mid_body,
    grid=(GRID,),
    in_specs=[_row_spec, _row_spec, _row_spec, _row_spec, _col_spec, _col_spec,
              _b_spec, _w_spec, _w_spec, _b_spec, _w_spec],
    out_specs=[_row_spec, _row_spec],
    out_shape=[jax.ShapeDtypeStruct((NPAD, D), jnp.float32),
               jax.ShapeDtypeStruct((NPAD, D), jnp.float32)],
)

_tc_post = pl.pallas_call(
    _tc_post_body,
    grid=(GRID,),
    in_specs=[_row_spec, _row_spec, _row_spec, _row_spec, _col_spec, _col_spec,
              _b_spec, _w_spec, _w_spec, _b_spec],
    out_specs=_row_spec,
    out_shape=jax.ShapeDtypeStruct((NPAD, D), jnp.float32),
)


def _pad_edges(edge_index):
    npad_e = EPTP - EPT
    src = edge_index[0].reshape(NT, EPT)
    dst = edge_index[1].reshape(NT, EPT)
    fill = jnp.arange(NT * npad_e, dtype=jnp.int32).reshape(NT, npad_e)
    src_fill = (fill * 97) % N
    dst_fill = N + (fill % (NPAD - N))
    src3 = jnp.concatenate([src, src_fill], axis=1).reshape(NT, CPT, CHUNK)
    dst3 = jnp.concatenate([dst, dst_fill], axis=1).reshape(NT, CPT, CHUNK)
    return src3, dst3


@jax.jit
def kernel(x, edge_index, edge_index_reverse, W1, bc1, W2, bc2,
           w11, w12, b1, w21, w22, b2):
    srcf3, dstf3 = _pad_edges(edge_index)
    srcr3, dstr3 = _pad_edges(edge_index_reverse)

    x_pad = jnp.zeros((NPAD, D), jnp.float32).at[:N].set(x)

    degf, degr = _deg_kernel(dstf3, dstr3)
    degf = degf.reshape(NPAD, 1)
    degr = degr.reshape(NPAD, 1)

    hs1f, hs1r, dinvf, dinvr = _tc_pre(x_pad, W1, degf, degr)
    agg1f, agg1r = _agg_kernel(hs1f, hs1r, srcf3, dstf3, srcr3, dstr3)
    hs2f, hs2r = _tc_mid(agg1f, agg1r, hs1f, hs1r, dinvf, dinvr,
                         bc1.reshape(1, D), w11.T, w12.T, b1.reshape(1, D), W2)
    agg2f, agg2r = _agg_kernel(hs2f, hs2r, srcf3, dstf3, srcr3, dstr3)
    out = _tc_post(agg2f, agg2r, hs2f, hs2r, dinvf, dinvr,
                   bc2.reshape(1, D), w21.T, w22.T, b2.reshape(1, D))
    return out[:N]


# speedup vs baseline: 25.4282x; 1.0608x over previous
import functools

import jax
import jax.numpy as jnp
from jax import lax
from jax.experimental import pallas as pl
from jax.experimental.pallas import tpu as pltpu
from jax.experimental.pallas import tpu_sc as plsc

N = 10000
D = 128
E = 320000

NT = 16
EPT = E // NT
CHUNK = 128
CPT = 160
EPTP = CPT * CHUNK
IG = 8
NG = CPT // IG
NPAD = 10240
RACC = NPAD // NT
BLK = 1024
GRID = NPAD // BLK

_mesh = plsc.VectorSubcoreMesh(core_axis_name="c", subcore_axis_name="s")


@functools.partial(
    pl.kernel,
    out_type=(jax.ShapeDtypeStruct((NPAD,), jnp.float32),
              jax.ShapeDtypeStruct((NPAD,), jnp.float32)),
    mesh=_mesh,
    scratch_types=(
        pltpu.VMEM((IG, CHUNK), jnp.int32),
        pltpu.VMEM((CHUNK,), jnp.float32),
        pltpu.VMEM_SHARED((NPAD,), jnp.float32),
    ),
)
def _deg_kernel(dstf, dstr, outf, outr, dst_v, ones_v, acc):
    cid = lax.axis_index("c")
    sid = lax.axis_index("s")

    def run(dst3, out):
        for k in range(CHUNK // 16):
            ones_v[pl.ds(k * 16, 16)] = jnp.zeros((16,), jnp.float32)
        base = sid * RACC
        for c in range(RACC // CHUNK):
            pltpu.sync_copy(ones_v, acc.at[pl.ds(base + c * CHUNK, CHUNK)])
        for k in range(CHUNK // 16):
            ones_v[pl.ds(k * 16, 16)] = jnp.ones((16,), jnp.float32)
        plsc.subcore_barrier()

        def group(g, carry):
            pltpu.sync_copy(dst3.at[sid, pl.ds(g * IG, IG)], dst_v)

            def body(j, c2):
                pltpu.sync_copy(ones_v, acc.at[dst_v.at[j]], add=True)
                return c2

            return lax.fori_loop(0, IG, body, carry)

        lax.fori_loop(0, NG, group, 0)
        plsc.subcore_barrier()
        pltpu.sync_copy(acc.at[pl.ds(base, RACC)], out.at[pl.ds(base, RACC)])

    @pl.when(cid == 0)
    def _():
        run(dstf, outf)

    @pl.when(cid == 1)
    def _():
        run(dstr, outr)


@functools.partial(
    pl.kernel,
    out_type=(jax.ShapeDtypeStruct((NPAD, D), jnp.float32),
              jax.ShapeDtypeStruct((NPAD, D), jnp.float32)),
    mesh=_mesh,
    scratch_types=(
        pltpu.VMEM((IG, CHUNK), jnp.int32),
        pltpu.VMEM((IG, CHUNK), jnp.int32),
        pltpu.VMEM((CHUNK, D), jnp.float32),
        pltpu.VMEM((CHUNK, D), jnp.float32),
        pltpu.VMEM_SHARED((NPAD, D), jnp.float32),
        pltpu.SemaphoreType.DMA,
        pltpu.SemaphoreType.DMA,
        pltpu.SemaphoreType.DMA,
    ),
)
def _agg_kernel(hsf, hsr, srcf, dstf, srcr, dstr, outf, outr,
                src_v, dst_v, rows0, rows1, acc, gsem, ssem0, ssem1):
    cid = lax.axis_index("c")
    sid = lax.axis_index("s")
    nbytes = CHUNK * D * 4

    def run(hs, src3, dst3, out):
        bufs = (rows0, rows1)
        ssems = (ssem0, ssem1)

        def wait_gather():
            pltpu.make_async_copy(hs.at[src_v.at[0]], rows0, gsem).wait()

        def wait_scatter(sem):
            pltpu.make_async_copy(rows1, acc.at[dst_v.at[0]], sem).wait()

        def zrow(r, carry):
            for k in range(D // 16):
                rows0[r, pl.ds(k * 16, 16)] = jnp.zeros((16,), jnp.float32)
                rows1[r, pl.ds(k * 16, 16)] = jnp.zeros((16,), jnp.float32)
            return carry

        lax.fori_loop(0, CHUNK, zrow, 0)
        base = sid * RACC
        for c in range(RACC // CHUNK):
            pltpu.sync_copy(rows0, acc.at[pl.ds(base + c * CHUNK, CHUNK)])
        plsc.subcore_barrier()

        pltpu.sync_copy(dst3.at[sid, pl.ds(0, IG)], dst_v)
        pltpu.sync_copy(src3.at[sid, pl.ds(0, IG)], src_v)
        pltpu.async_copy(rows1, acc.at[dst_v.at[0]], ssem1, add=True)
        pltpu.async_copy(hs.at[src_v.at[0]], rows0, gsem)

        def group(g, carry):
            wait_scatter(ssem1)
            pltpu.sync_copy(dst3.at[sid, pl.ds(g * IG, IG)], dst_v)
            gd = [None] * IG
            sd = [None] * IG
            for j in range(IG):
                b = bufs[j % 2]
                if j == 0:
                    wait_gather()
                else:
                    gd[j].wait()
                sd[j] = pltpu.async_copy(b, acc.at[dst_v.at[j]], ssems[j % 2],
                                         add=True)
                if j + 1 < IG:
                    if j >= 1:
                        sd[j - 1].wait()
                    gd[j + 1] = pltpu.async_copy(hs.at[src_v.at[j + 1]],
                                                 bufs[(j + 1) % 2], gsem)
            sd[IG - 2].wait()
            pltpu.sync_copy(src3.at[sid, pl.ds((g + 1) * IG, IG)], src_v)
            pltpu.async_copy(hs.at[src_v.at[0]], rows0, gsem)
            return carry

        lax.fori_loop(0, NG, group, 0)
        wait_gather()
        wait_scatter(ssem1)
        plsc.subcore_barrier()
        pltpu.sync_copy(acc.at[pl.ds(base, RACC)], out.at[pl.ds(base, RACC)])

    @pl.when(cid == 0)
    def _():
        run(hsf, srcf, dstf, outf)

    @pl.when(cid == 1)
    def _():
        run(hsr, srcr, dstr, outr)


def _tc_pre_body(x_ref, w1_ref, degf_ref, degr_ref,
                 hsf_ref, hsr_ref, dinvf_ref, dinvr_ref):
    h1 = jnp.dot(x_ref[...], w1_ref[...], preferred_element_type=jnp.float32)
    dinvf = lax.rsqrt(degf_ref[...] + 1.0)
    dinvr = lax.rsqrt(degr_ref[...] + 1.0)
    hsf_ref[...] = h1 * dinvf
    hsr_ref[...] = h1 * dinvr
    dinvf_ref[...] = dinvf
    dinvr_ref[...] = dinvr


def _tc_mid_body(aggf_ref, aggr_ref, hsf_ref, hsr_ref, dinvf_ref, dinvr_ref,
                 bc1_ref, w11t_ref, w12t_ref, b1_ref, w2_ref,
                 hs2f_ref, hs2r_ref):
    dinvf = dinvf_ref[...]
    dinvr = dinvr_ref[...]
    c11 = jax.nn.relu(dinvf * (aggf_ref[...] + hsf_ref[...]) + bc1_ref[...])
    c12 = jax.nn.relu(dinvr * (aggr_ref[...] + hsr_ref[...]) + bc1_ref[...])
    g = jax.nn.sigmoid(
        jnp.dot(c11, w11t_ref[...], preferred_element_type=jnp.float32)
        + jnp.dot(c12, w12t_ref[...], preferred_element_type=jnp.float32)
        + b1_ref[...])
    h = g * c11 + (1.0 - g) * c12
    h2 = jnp.dot(h, w2_ref[...], preferred_element_type=jnp.float32)
    hs2f_ref[...] = h2 * dinvf
    hs2r_ref[...] = h2 * dinvr


def _tc_post_body(aggf_ref, aggr_ref, hsf_ref, hsr_ref, dinvf_ref, dinvr_ref,
                  bc2_ref, w21t_ref, w22t_ref, b2_ref, out_ref):
    c21 = jax.nn.relu(dinvf_ref[...] * (aggf_ref[...] + hsf_ref[...]) + bc2_ref[...])
    c22 = jax.nn.relu(dinvr_ref[...] * (aggr_ref[...] + hsr_ref[...]) + bc2_ref[...])
    g2 = jax.nn.sigmoid(
        jnp.dot(c21, w21t_ref[...], preferred_element_type=jnp.float32)
        + jnp.dot(c22, w22t_ref[...], preferred_element_type=jnp.float32)
        + b2_ref[...])
    out_ref[...] = g2 * c21 + (1.0 - g2) * c22


_row_spec = pl.BlockSpec((BLK, D), lambda b: (b, 0))
_col_spec = pl.BlockSpec((BLK, 1), lambda b: (b, 0))
_w_spec = pl.BlockSpec((D, D), lambda b: (0, 0))
_b_spec = pl.BlockSpec((1, D), lambda b: (0, 0))

_tc_pre = pl.pallas_call(
    _tc_pre_body,
    grid=(GRID,),
    in_specs=[_row_spec, _w_spec, _col_spec, _col_spec],
    out_specs=[_row_spec, _row_spec, _col_spec, _col_spec],
    out_shape=[jax.ShapeDtypeStruct((NPAD, D), jnp.float32),
               jax.ShapeDtypeStruct((NPAD, D), jnp.float32),
               jax.ShapeDtypeStruct((NPAD, 1), jnp.float32),
               jax.ShapeDtypeStruct((NPAD, 1), jnp.float32)],
)

_tc_mid = pl.pallas_call(
    _tc_mid_body,
    grid=(GRID,),
    in_specs=[_row_spec, _row_spec, _row_spec, _row_spec, _col_spec, _col_spec,
              _b_spec, _w_spec, _w_spec, _b_spec, _w_spec],
    out_specs=[_row_spec, _row_spec],
    out_shape=[jax.ShapeDtypeStruct((NPAD, D), jnp.float32),
               jax.ShapeDtypeStruct((NPAD, D), jnp.float32)],
)

_tc_post = pl.pallas_call(
    _tc_post_body,
    grid=(GRID,),
    in_specs=[_row_spec, _row_spec, _row_spec, _row_spec, _col_spec, _col_spec,
              _b_spec, _w_spec, _w_spec, _b_spec],
    out_specs=_row_spec,
    out_shape=jax.ShapeDtypeStruct((NPAD, D), jnp.float32),
)


def _pad_edges(edge_index):
    npad_e = EPTP - EPT
    src = edge_index[0].reshape(NT, EPT)
    dst = edge_index[1].reshape(NT, EPT)
    fill = jnp.arange(NT * npad_e, dtype=jnp.int32).reshape(NT, npad_e)
    src_fill = (fill * 97) % N
    dst_fill = N + (fill % (NPAD - N))
    src3 = jnp.concatenate([src, src_fill], axis=1).reshape(NT, CPT, CHUNK)
    dst3 = jnp.concatenate([dst, dst_fill], axis=1).reshape(NT, CPT, CHUNK)
    extra = ((jnp.arange(NT * IG * CHUNK, dtype=jnp.int32) * 131) % N
             ).reshape(NT, IG, CHUNK)
    src3 = jnp.concatenate([src3, extra], axis=1)
    return src3, dst3


@jax.jit
def kernel(x, edge_index, edge_index_reverse, W1, bc1, W2, bc2,
           w11, w12, b1, w21, w22, b2):
    srcf3, dstf3 = _pad_edges(edge_index)
    srcr3, dstr3 = _pad_edges(edge_index_reverse)

    x_pad = jnp.zeros((NPAD, D), jnp.float32).at[:N].set(x)

    degf, degr = _deg_kernel(dstf3, dstr3)
    degf = degf.reshape(NPAD, 1)
    degr = degr.reshape(NPAD, 1)

    hs1f, hs1r, dinvf, dinvr = _tc_pre(x_pad, W1, degf, degr)
    agg1f, agg1r = _agg_kernel(hs1f, hs1r, srcf3, dstf3, srcr3, dstr3)
    hs2f, hs2r = _tc_mid(agg1f, agg1r, hs1f, hs1r, dinvf, dinvr,
                         bc1.reshape(1, D), w11.T, w12.T, b1.reshape(1, D), W2)
    agg2f, agg2r = _agg_kernel(hs2f, hs2r, srcf3, dstf3, srcr3, dstr3)
    out = _tc_post(agg2f, agg2r, hs2f, hs2r, dinvf, dinvr,
                   bc2.reshape(1, D), w21.T, w22.T, b2.reshape(1, D))
    return out[:N]


# speedup vs baseline: 25.9195x; 1.0193x over previous
import functools

import jax
import jax.numpy as jnp
from jax import lax
from jax.experimental import pallas as pl
from jax.experimental.pallas import tpu as pltpu
from jax.experimental.pallas import tpu_sc as plsc

N = 10000
D = 128
E = 320000

NT = 16
EPT = E // NT
CHUNK = 128
CPT = 160
EPTP = CPT * CHUNK
IG = 8
NG = CPT // IG
NPAD = 10240
RACC = NPAD // NT
BLK = 1024
GRID = NPAD // BLK

_mesh = plsc.VectorSubcoreMesh(core_axis_name="c", subcore_axis_name="s")


@functools.partial(
    pl.kernel,
    out_type=(jax.ShapeDtypeStruct((NPAD,), jnp.float32),
              jax.ShapeDtypeStruct((NPAD,), jnp.float32)),
    mesh=_mesh,
    scratch_types=(
        pltpu.VMEM((IG, CHUNK), jnp.int32),
        pltpu.VMEM((CHUNK,), jnp.float32),
        pltpu.VMEM_SHARED((NPAD,), jnp.float32),
        pltpu.SemaphoreType.DMA,
    ),
)
def _deg_kernel(dstf, dstr, outf, outr, dst_v, ones_v, acc, ssem):
    cid = lax.axis_index("c")
    sid = lax.axis_index("s")

    def run(dst3, out):
        for k in range(CHUNK // 16):
            ones_v[pl.ds(k * 16, 16)] = jnp.zeros((16,), jnp.float32)
        base = sid * RACC
        for c in range(RACC // CHUNK):
            pltpu.sync_copy(ones_v, acc.at[pl.ds(base + c * CHUNK, CHUNK)])
        for k in range(CHUNK // 16):
            ones_v[pl.ds(k * 16, 16)] = jnp.ones((16,), jnp.float32)
        plsc.subcore_barrier()

        def group(g, carry):
            pltpu.sync_copy(dst3.at[sid, pl.ds(g * IG, IG)], dst_v)
            sd = [pltpu.async_copy(ones_v, acc.at[dst_v.at[j]], ssem, add=True)
                  for j in range(IG)]
            for d in sd:
                d.wait()
            return carry

        lax.fori_loop(0, NG, group, 0)
        plsc.subcore_barrier()
        pltpu.sync_copy(acc.at[pl.ds(base, RACC)], out.at[pl.ds(base, RACC)])

    @pl.when(cid == 0)
    def _():
        run(dstf, outf)

    @pl.when(cid == 1)
    def _():
        run(dstr, outr)


@functools.partial(
    pl.kernel,
    out_type=(jax.ShapeDtypeStruct((NPAD, D), jnp.float32),
              jax.ShapeDtypeStruct((NPAD, D), jnp.float32)),
    mesh=_mesh,
    scratch_types=(
        pltpu.VMEM((IG, CHUNK), jnp.int32),
        pltpu.VMEM((IG, CHUNK), jnp.int32),
        pltpu.VMEM((CHUNK, D), jnp.float32),
        pltpu.VMEM((CHUNK, D), jnp.float32),
        pltpu.VMEM_SHARED((NPAD, D), jnp.float32),
        pltpu.SemaphoreType.DMA,
        pltpu.SemaphoreType.DMA,
        pltpu.SemaphoreType.DMA,
    ),
)
def _agg_kernel(hsf, hsr, srcf, dstf, srcr, dstr, outf, outr,
                src_v, dst_v, rows0, rows1, acc, gsem, ssem0, ssem1):
    cid = lax.axis_index("c")
    sid = lax.axis_index("s")
    nbytes = CHUNK * D * 4

    def run(hs, src3, dst3, out):
        bufs = (rows0, rows1)
        ssems = (ssem0, ssem1)

        def wait_gather():
            pltpu.make_async_copy(hs.at[src_v.at[0]], rows0, gsem).wait()

        def wait_scatter(sem):
            pltpu.make_async_copy(rows1, acc.at[dst_v.at[0]], sem).wait()

        def zrow(r, carry):
            for k in range(D // 16):
                rows0[r, pl.ds(k * 16, 16)] = jnp.zeros((16,), jnp.float32)
                rows1[r, pl.ds(k * 16, 16)] = jnp.zeros((16,), jnp.float32)
            return carry

        lax.fori_loop(0, CHUNK, zrow, 0)
        base = sid * RACC
        for c in range(RACC // CHUNK):
            pltpu.sync_copy(rows0, acc.at[pl.ds(base + c * CHUNK, CHUNK)])
        plsc.subcore_barrier()

        pltpu.sync_copy(dst3.at[sid, pl.ds(0, IG)], dst_v)
        pltpu.sync_copy(src3.at[sid, pl.ds(0, IG)], src_v)
        pltpu.async_copy(rows1, acc.at[dst_v.at[0]], ssem1, add=True)
        pltpu.async_copy(hs.at[src_v.at[0]], rows0, gsem)

        def group(g, carry):
            wait_scatter(ssem1)
            pltpu.sync_copy(dst3.at[sid, pl.ds(g * IG, IG)], dst_v)
            gd = [None] * IG
            sd = [None] * IG
            for j in range(IG):
                b = bufs[j % 2]
                if j == 0:
                    wait_gather()
                else:
                    gd[j].wait()
                sd[j] = pltpu.async_copy(b, acc.at[dst_v.at[j]], ssems[j % 2],
                                         add=True)
                if j + 1 < IG:
                    if j >= 1:
                        sd[j - 1].wait()
                    gd[j + 1] = pltpu.async_copy(hs.at[src_v.at[j + 1]],
                                                 bufs[(j + 1) % 2], gsem)
            sd[IG - 2].wait()
            pltpu.sync_copy(src3.at[sid, pl.ds((g + 1) * IG, IG)], src_v)
            pltpu.async_copy(hs.at[src_v.at[0]], rows0, gsem)
            return carry

        lax.fori_loop(0, NG, group, 0)
        wait_gather()
        wait_scatter(ssem1)
        plsc.subcore_barrier()
        pltpu.sync_copy(acc.at[pl.ds(base, RACC)], out.at[pl.ds(base, RACC)])

    @pl.when(cid == 0)
    def _():
        run(hsf, srcf, dstf, outf)

    @pl.when(cid == 1)
    def _():
        run(hsr, srcr, dstr, outr)


def _tc_pre_body(x_ref, w1_ref, degf_ref, degr_ref,
                 hsf_ref, hsr_ref, dinvf_ref, dinvr_ref):
    h1 = jnp.dot(x_ref[...], w1_ref[...], preferred_element_type=jnp.float32)
    dinvf = lax.rsqrt(degf_ref[...] + 1.0)
    dinvr = lax.rsqrt(degr_ref[...] + 1.0)
    hsf_ref[...] = h1 * dinvf
    hsr_ref[...] = h1 * dinvr
    dinvf_ref[...] = dinvf
    dinvr_ref[...] = dinvr


def _tc_mid_body(aggf_ref, aggr_ref, hsf_ref, hsr_ref, dinvf_ref, dinvr_ref,
                 bc1_ref, w11t_ref, w12t_ref, b1_ref, w2_ref,
                 hs2f_ref, hs2r_ref):
    dinvf = dinvf_ref[...]
    dinvr = dinvr_ref[...]
    c11 = jax.nn.relu(dinvf * (aggf_ref[...] + hsf_ref[...]) + bc1_ref[...])
    c12 = jax.nn.relu(dinvr * (aggr_ref[...] + hsr_ref[...]) + bc1_ref[...])
    g = jax.nn.sigmoid(
        jnp.dot(c11, w11t_ref[...], preferred_element_type=jnp.float32)
        + jnp.dot(c12, w12t_ref[...], preferred_element_type=jnp.float32)
        + b1_ref[...])
    h = g * c11 + (1.0 - g) * c12
    h2 = jnp.dot(h, w2_ref[...], preferred_element_type=jnp.float32)
    hs2f_ref[...] = h2 * dinvf
    hs2r_ref[...] = h2 * dinvr


def _tc_post_body(aggf_ref, aggr_ref, hsf_ref, hsr_ref, dinvf_ref, dinvr_ref,
                  bc2_ref, w21t_ref, w22t_ref, b2_ref, out_ref):
    c21 = jax.nn.relu(dinvf_ref[...] * (aggf_ref[...] + hsf_ref[...]) + bc2_ref[...])
    c22 = jax.nn.relu(dinvr_ref[...] * (aggr_ref[...] + hsr_ref[...]) + bc2_ref[...])
    g2 = jax.nn.sigmoid(
        jnp.dot(c21, w21t_ref[...], preferred_element_type=jnp.float32)
        + jnp.dot(c22, w22t_ref[...], preferred_element_type=jnp.float32)
        + b2_ref[...])
    out_ref[...] = g2 * c21 + (1.0 - g2) * c22


_row_spec = pl.BlockSpec((BLK, D), lambda b: (b, 0))
_col_spec = pl.BlockSpec((BLK, 1), lambda b: (b, 0))
_w_spec = pl.BlockSpec((D, D), lambda b: (0, 0))
_b_spec = pl.BlockSpec((1, D), lambda b: (0, 0))

_tc_pre = pl.pallas_call(
    _tc_pre_body,
    grid=(GRID,),
    in_specs=[_row_spec, _w_spec, _col_spec, _col_spec],
    out_specs=[_row_spec, _row_spec, _col_spec, _col_spec],
    out_shape=[jax.ShapeDtypeStruct((N, D), jnp.float32),
               jax.ShapeDtypeStruct((N, D), jnp.float32),
               jax.ShapeDtypeStruct((NPAD, 1), jnp.float32),
               jax.ShapeDtypeStruct((NPAD, 1), jnp.float32)],
)

_tc_mid = pl.pallas_call(
    _tc_mid_body,
    grid=(GRID,),
    in_specs=[_row_spec, _row_spec, _row_spec, _row_spec, _col_spec, _col_spec,
              _b_spec, _w_spec, _w_spec, _b_spec, _w_spec],
    out_specs=[_row_spec, _row_spec],
    out_shape=[jax.ShapeDtypeStruct((N, D), jnp.float32),
               jax.ShapeDtypeStruct((N, D), jnp.float32)],
)

_tc_post = pl.pallas_call(
    _tc_post_body,
    grid=(GRID,),
    in_specs=[_row_spec, _row_spec, _row_spec, _row_spec, _col_spec, _col_spec,
              _b_spec, _w_spec, _w_spec, _b_spec],
    out_specs=_row_spec,
    out_shape=jax.ShapeDtypeStruct((N, D), jnp.float32),
)


def _pad_edges(edge_index):
    npad_e = EPTP - EPT
    src = edge_index[0].reshape(NT, EPT)
    dst = edge_index[1].reshape(NT, EPT)
    fill = jnp.arange(NT * npad_e, dtype=jnp.int32).reshape(NT, npad_e)
    src_fill = (fill * 97) % N
    dst_fill = N + (fill % (NPAD - N))
    src3 = jnp.concatenate([src, src_fill], axis=1).reshape(NT, CPT, CHUNK)
    dst3 = jnp.concatenate([dst, dst_fill], axis=1).reshape(NT, CPT, CHUNK)
    extra = ((jnp.arange(NT * IG * CHUNK, dtype=jnp.int32) * 131) % N
             ).reshape(NT, IG, CHUNK)
    src3 = jnp.concatenate([src3, extra], axis=1)
    return src3, dst3


@jax.jit
def kernel(x, edge_index, edge_index_reverse, W1, bc1, W2, bc2,
           w11, w12, b1, w21, w22, b2):
    srcf3, dstf3 = _pad_edges(edge_index)
    srcr3, dstr3 = _pad_edges(edge_index_reverse)

    degf, degr = _deg_kernel(dstf3, dstr3)
    degf = degf.reshape(NPAD, 1)
    degr = degr.reshape(NPAD, 1)

    hs1f, hs1r, dinvf, dinvr = _tc_pre(x, W1, degf, degr)
    agg1f, agg1r = _agg_kernel(hs1f, hs1r, srcf3, dstf3, srcr3, dstr3)
    hs2f, hs2r = _tc_mid(agg1f, agg1r, hs1f, hs1r, dinvf, dinvr,
                         bc1.reshape(1, D), w11.T, w12.T, b1.reshape(1, D), W2)
    agg2f, agg2r = _agg_kernel(hs2f, hs2r, srcf3, dstf3, srcr3, dstr3)
    return _tc_post(agg2f, agg2r, hs2f, hs2r, dinvf, dinvr,
                    bc2.reshape(1, D), w21.T, w22.T, b2.reshape(1, D))


# speedup vs baseline: 28.8189x; 1.1119x over previous
import functools

import jax
import jax.numpy as jnp
from jax import lax
from jax.experimental import pallas as pl
from jax.experimental.pallas import tpu as pltpu
from jax.experimental.pallas import tpu_sc as plsc

N = 10000
D = 128
E = 320000

NT = 16
EPT = E // NT
CHUNK = 80
CPT = 256
EPTP = CPT * CHUNK
IG = 8
NG = CPT // IG
NBUF = 4
NPAD = 10240
RACC = NPAD // NT
BLK = 1024
GRID = NPAD // BLK

_mesh = plsc.VectorSubcoreMesh(core_axis_name="c", subcore_axis_name="s")


@functools.partial(
    pl.kernel,
    out_type=(jax.ShapeDtypeStruct((NPAD,), jnp.float32),
              jax.ShapeDtypeStruct((NPAD,), jnp.float32)),
    mesh=_mesh,
    scratch_types=(
        pltpu.VMEM((IG, CHUNK), jnp.int32),
        pltpu.VMEM((CHUNK,), jnp.float32),
        pltpu.VMEM_SHARED((NPAD,), jnp.float32),
        pltpu.SemaphoreType.DMA,
    ),
)
def _deg_kernel(dstf, dstr, outf, outr, dst_v, ones_v, acc, ssem):
    cid = lax.axis_index("c")
    sid = lax.axis_index("s")

    def run(dst3, out):
        for k in range(CHUNK // 16):
            ones_v[pl.ds(k * 16, 16)] = jnp.zeros((16,), jnp.float32)
        base = sid * RACC
        for c in range(RACC // CHUNK):
            pltpu.sync_copy(ones_v, acc.at[pl.ds(base + c * CHUNK, CHUNK)])
        for k in range(CHUNK // 16):
            ones_v[pl.ds(k * 16, 16)] = jnp.ones((16,), jnp.float32)
        plsc.subcore_barrier()

        def group(g, carry):
            pltpu.sync_copy(dst3.at[sid, pl.ds(g * IG, IG)], dst_v)
            sd = [pltpu.async_copy(ones_v, acc.at[dst_v.at[j]], ssem, add=True)
                  for j in range(IG)]
            for d in sd:
                d.wait()
            return carry

        lax.fori_loop(0, NG, group, 0)
        plsc.subcore_barrier()
        pltpu.sync_copy(acc.at[pl.ds(base, RACC)], out.at[pl.ds(base, RACC)])

    @pl.when(cid == 0)
    def _():
        run(dstf, outf)

    @pl.when(cid == 1)
    def _():
        run(dstr, outr)


@functools.partial(
    pl.kernel,
    out_type=(jax.ShapeDtypeStruct((NPAD, D), jnp.float32),
              jax.ShapeDtypeStruct((NPAD, D), jnp.float32)),
    mesh=_mesh,
    scratch_types=(
        pltpu.VMEM((2 * IG, CHUNK), jnp.int32),
        pltpu.VMEM((IG, CHUNK), jnp.int32),
        pltpu.VMEM((CHUNK, D), jnp.float32),
        pltpu.VMEM((CHUNK, D), jnp.float32),
        pltpu.VMEM((CHUNK, D), jnp.float32),
        pltpu.VMEM((CHUNK, D), jnp.float32),
        pltpu.VMEM_SHARED((NPAD, D), jnp.float32),
        pltpu.SemaphoreType.DMA,
        pltpu.SemaphoreType.DMA,
        pltpu.SemaphoreType.DMA,
        pltpu.SemaphoreType.DMA,
        pltpu.SemaphoreType.DMA,
        pltpu.SemaphoreType.DMA,
        pltpu.SemaphoreType.DMA,
        pltpu.SemaphoreType.DMA,
    ),
)
def _agg_kernel(hsf, hsr, srcf, dstf, srcr, dstr, outf, outr,
                src_v, dst_v, rows0, rows1, rows2, rows3, acc,
                gsem0, gsem1, gsem2, gsem3, ssem0, ssem1, ssem2, ssem3):
    cid = lax.axis_index("c")
    sid = lax.axis_index("s")

    def run(hs, src3, dst3, out):
        bufs = (rows0, rows1, rows2, rows3)
        gsems = (gsem0, gsem1, gsem2, gsem3)
        ssems = (ssem0, ssem1, ssem2, ssem3)

        def wait_gather(b):
            pltpu.make_async_copy(hs.at[src_v.at[0]], bufs[b], gsems[b]).wait()

        def wait_scatter(b):
            pltpu.make_async_copy(bufs[b], acc.at[dst_v.at[0]],
                                  ssems[b]).wait()

        def zrow(r, carry):
            for k in range(D // 16):
                for b in bufs:
                    b[r, pl.ds(k * 16, 16)] = jnp.zeros((16,), jnp.float32)
            return carry

        lax.fori_loop(0, CHUNK, zrow, 0)
        base = sid * RACC
        for c in range(RACC // CHUNK):
            pltpu.sync_copy(rows0, acc.at[pl.ds(base + c * CHUNK, CHUNK)])
        plsc.subcore_barrier()

        pltpu.sync_copy(dst3.at[sid, pl.ds(0, IG)], dst_v)
        pltpu.sync_copy(src3.at[sid, pl.ds(0, IG)], src_v.at[pl.ds(IG, IG)])
        pltpu.async_copy(rows3, acc.at[dst_v.at[0]], ssem3, add=True)
        for b in range(NBUF - 1):
            pltpu.async_copy(hs.at[src_v.at[IG + b]], bufs[b], gsems[b])

        def group(g, carry):
            wait_scatter(3)
            pltpu.sync_copy(dst3.at[sid, pl.ds(g * IG, IG)], dst_v)
            pltpu.sync_copy(src3.at[sid, pl.ds(g * IG, IG)],
                            src_v.at[pl.ds(0, IG)])
            sd = [None] * IG
            for j in range(IG):
                b = j % NBUF
                wait_gather(b)
                sd[j] = pltpu.async_copy(bufs[b], acc.at[dst_v.at[j]],
                                         ssems[b], add=True)
                if j == IG - 3:
                    pltpu.sync_copy(src3.at[sid, pl.ds((g + 1) * IG, IG)],
                                    src_v.at[pl.ds(IG, IG)])
                if j > 0:
                    wait_scatter((j + NBUF - 1) % NBUF)
                nxt = j + NBUF - 1
                pltpu.async_copy(hs.at[src_v.at[nxt]], bufs[nxt % NBUF],
                                 gsems[nxt % NBUF])
            return carry

        lax.fori_loop(0, NG, group, 0)
        for b in range(NBUF - 1):
            wait_gather(b)
        wait_scatter(3)
        plsc.subcore_barrier()
        pltpu.sync_copy(acc.at[pl.ds(base, RACC)], out.at[pl.ds(base, RACC)])

    @pl.when(cid == 0)
    def _():
        run(hsf, srcf, dstf, outf)

    @pl.when(cid == 1)
    def _():
        run(hsr, srcr, dstr, outr)


def _tc_pre_body(x_ref, w1_ref, degf_ref, degr_ref,
                 hsf_ref, hsr_ref, dinvf_ref, dinvr_ref):
    h1 = jnp.dot(x_ref[...], w1_ref[...], preferred_element_type=jnp.float32)
    dinvf = lax.rsqrt(degf_ref[...] + 1.0)
    dinvr = lax.rsqrt(degr_ref[...] + 1.0)
    hsf_ref[...] = h1 * dinvf
    hsr_ref[...] = h1 * dinvr
    dinvf_ref[...] = dinvf
    dinvr_ref[...] = dinvr


def _tc_mid_body(aggf_ref, aggr_ref, hsf_ref, hsr_ref, dinvf_ref, dinvr_ref,
                 bc1_ref, w11t_ref, w12t_ref, b1_ref, w2_ref,
                 hs2f_ref, hs2r_ref):
    dinvf = dinvf_ref[...]
    dinvr = dinvr_ref[...]
    c11 = jax.nn.relu(dinvf * (aggf_ref[...] + hsf_ref[...]) + bc1_ref[...])
    c12 = jax.nn.relu(dinvr * (aggr_ref[...] + hsr_ref[...]) + bc1_ref[...])
    g = jax.nn.sigmoid(
        jnp.dot(c11, w11t_ref[...], preferred_element_type=jnp.float32)
        + jnp.dot(c12, w12t_ref[...], preferred_element_type=jnp.float32)
        + b1_ref[...])
    h = g * c11 + (1.0 - g) * c12
    h2 = jnp.dot(h, w2_ref[...], preferred_element_type=jnp.float32)
    hs2f_ref[...] = h2 * dinvf
    hs2r_ref[...] = h2 * dinvr


def _tc_post_body(aggf_ref, aggr_ref, hsf_ref, hsr_ref, dinvf_ref, dinvr_ref,
                  bc2_ref, w21t_ref, w22t_ref, b2_ref, out_ref):
    c21 = jax.nn.relu(dinvf_ref[...] * (aggf_ref[...] + hsf_ref[...]) + bc2_ref[...])
    c22 = jax.nn.relu(dinvr_ref[...] * (aggr_ref[...] + hsr_ref[...]) + bc2_ref[...])
    g2 = jax.nn.sigmoid(
        jnp.dot(c21, w21t_ref[...], preferred_element_type=jnp.float32)
        + jnp.dot(c22, w22t_ref[...], preferred_element_type=jnp.float32)
        + b2_ref[...])
    out_ref[...] = g2 * c21 + (1.0 - g2) * c22


_row_spec = pl.BlockSpec((BLK, D), lambda b: (b, 0))
_col_spec = pl.BlockSpec((BLK, 1), lambda b: (b, 0))
_w_spec = pl.BlockSpec((D, D), lambda b: (0, 0))
_b_spec = pl.BlockSpec((1, D), lambda b: (0, 0))

_tc_pre = pl.pallas_call(
    _tc_pre_body,
    grid=(GRID,),
    in_specs=[_row_spec, _w_spec, _col_spec, _col_spec],
    out_specs=[_row_spec, _row_spec, _col_spec, _col_spec],
    out_shape=[jax.ShapeDtypeStruct((N, D), jnp.float32),
               jax.ShapeDtypeStruct((N, D), jnp.float32),
               jax.ShapeDtypeStruct((NPAD, 1), jnp.float32),
               jax.ShapeDtypeStruct((NPAD, 1), jnp.float32)],
)

_tc_mid = pl.pallas_call(
    _tc_mid_body,
    grid=(GRID,),
    in_specs=[_row_spec, _row_spec, _row_spec, _row_spec, _col_spec, _col_spec,
              _b_spec, _w_spec, _w_spec, _b_spec, _w_spec],
    out_specs=[_row_spec, _row_spec],
    out_shape=[jax.ShapeDtypeStruct((N, D), jnp.float32),
               jax.ShapeDtypeStruct((N, D), jnp.float32)],
)

_tc_post = pl.pallas_call(
    _tc_post_body,
    grid=(GRID,),
    in_specs=[_row_spec, _row_spec, _row_spec, _row_spec, _col_spec, _col_spec,
              _b_spec, _w_spec, _w_spec, _b_spec],
    out_specs=_row_spec,
    out_shape=jax.ShapeDtypeStruct((N, D), jnp.float32),
)


def _pad_edges(edge_index):
    npad_e = EPTP - EPT
    src = edge_index[0].reshape(NT, EPT)
    dst = edge_index[1].reshape(NT, EPT)
    fill = jnp.arange(NT * npad_e, dtype=jnp.int32).reshape(NT, npad_e)
    src_fill = (fill * 97) % N
    dst_fill = N + (fill % (NPAD - N))
    src3 = jnp.concatenate([src, src_fill], axis=1).reshape(NT, CPT, CHUNK)
    dst3 = jnp.concatenate([dst, dst_fill], axis=1).reshape(NT, CPT, CHUNK)
    extra = ((jnp.arange(NT * IG * CHUNK, dtype=jnp.int32) * 131) % N
             ).reshape(NT, IG, CHUNK)
    src3 = jnp.concatenate([src3, extra], axis=1)
    return src3, dst3


@jax.jit
def kernel(x, edge_index, edge_index_reverse, W1, bc1, W2, bc2,
           w11, w12, b1, w21, w22, b2):
    srcf3, dstf3 = _pad_edges(edge_index)
    srcr3, dstr3 = _pad_edges(edge_index_reverse)

    degf, degr = _deg_kernel(dstf3, dstr3)
    degf = degf.reshape(NPAD, 1)
    degr = degr.reshape(NPAD, 1)

    hs1f, hs1r, dinvf, dinvr = _tc_pre(x, W1, degf, degr)
    agg1f, agg1r = _agg_kernel(hs1f, hs1r, srcf3, dstf3, srcr3, dstr3)
    hs2f, hs2r = _tc_mid(agg1f, agg1r, hs1f, hs1r, dinvf, dinvr,
                         bc1.reshape(1, D), w11.T, w12.T, b1.reshape(1, D), W2)
    agg2f, agg2r = _agg_kernel(hs2f, hs2r, srcf3, dstf3, srcr3, dstr3)
    return _tc_post(agg2f, agg2r, hs2f, hs2r, dinvf, dinvr,
                    bc2.reshape(1, D), w21.T, w22.T, b2.reshape(1, D))
